# baseline (speedup 1.0000x reference)
import functools

import jax
import jax.numpy as jnp
from jax import lax
from jax.experimental import pallas as pl
from jax.experimental.pallas import tpu as pltpu
from jax.experimental.pallas import tpu_sc as plsc

_NC = 2
_NS = 16
_L = 16
_NW = _NC * _NS

_MULT_CUR = 36313
_MULT_PREV = 27191
_CHUNK = 128


def _make_sc_kernel(N, V, D, per_w):
    n_chunks = per_w // _CHUNK
    pairs_w = per_w // 2
    n_groups = per_w // _L
    W = 2 * D
    mesh = plsc.VectorSubcoreMesh(core_axis_name="c", subcore_axis_name="s")

    @functools.partial(
        pl.kernel,
        out_type=jax.ShapeDtypeStruct((N // 2, W), jnp.float32),
        mesh=mesh,
        scratch_types=[
            pltpu.VMEM((per_w,), jnp.int32),
            pltpu.VMEM((per_w,), jnp.int32),
            pltpu.VMEM((per_w,), jnp.int32),
            pltpu.VMEM((n_groups, _L), jnp.int32),
            pltpu.VMEM((per_w, W), jnp.float32),
            pltpu.VMEM((pairs_w, W), jnp.float32),
            pltpu.VMEM((_L,), jnp.float32),
            pltpu.SemaphoreType.DMA,
            pltpu.SemaphoreType.DMA,
            pltpu.SemaphoreType.DMA,
        ],
        compiler_params=pltpu.CompilerParams(needs_layout_passes=False),
    )
    def sc_kernel(t_hbm, p_hbm, s_hbm, embed_hbm, out_hbm,
                  t_v, p_v, idx_v, off_v, rows_v, out_v, s_v,
                  sem_in, sem_g, sem_out):
        wid = lax.axis_index("s") * _NC + lax.axis_index("c")
        base = wid * per_w
        pairs_base = wid * pairs_w

        cp_t = pltpu.async_copy(t_hbm.at[pl.ds(base, per_w)], t_v, sem_in)
        cp_p = pltpu.async_copy(p_hbm.at[pl.ds(base, per_w)], p_v, sem_in)
        pltpu.sync_copy(s_hbm, s_v)
        cp_t.wait()
        cp_p.wait()

        for j in range(n_groups):
            cur = t_v[pl.ds(j * _L, _L)]
            prv = p_v[pl.ds(j * _L, _L)]
            h = lax.bitwise_xor(cur * _MULT_CUR, prv * _MULT_PREV) % (V - 1)
            idx_v[pl.ds(j * _L, _L)] = h >> 1
            off_v[j] = (h & 1) * D

        gathers = []
        for c in range(n_chunks):
            gathers.append(pltpu.async_copy(
                embed_hbm.at[idx_v.at[pl.ds(c * _CHUNK, _CHUNK)]],
                rows_v.at[pl.ds(c * _CHUNK, _CHUNK)],
                sem_g))

        sv = s_v[...]
        iota = lax.iota(jnp.int32, _L)
        dst_half = (iota & 1) * D
        groups_per_chunk = _CHUNK // _L
        stores = []
        for c in range(n_chunks):
            gathers[c].wait()

            def select_group(g, _, c=c):
                gi = c * groups_per_chunk + g
                rvec = gi * _L + iota
                qvec = lax.shift_right_logical(rvec, 1)
                offs = off_v[gi]
                for cc in range(D):
                    v = plsc.load_gather(rows_v, [rvec, offs + cc])
                    plsc.store_scatter(out_v, [qvec, dst_half + cc], v * sv)
                return 0

            lax.fori_loop(0, groups_per_chunk, select_group, 0)
            stores.append(pltpu.async_copy(
                out_v.at[pl.ds(c * _CHUNK // 2, _CHUNK // 2)],
                out_hbm.at[pl.ds(pairs_base + c * _CHUNK // 2, _CHUNK // 2)],
                sem_out))
        for cp in stores:
            cp.wait()

    return sc_kernel


def kernel(x, embed, scale):
    B, S = x.shape
    V, D = embed.shape
    N = B * S
    per_w = N // _NW

    t = x.astype(jnp.int32)
    prev = jnp.concatenate([jnp.zeros_like(t[:, :1]), t[:, :-1]], axis=1)
    scale_vec = jnp.full((_L,), scale, jnp.float32)
    embed_pairs = embed.reshape(V // 2, 2 * D)

    sc = _make_sc_kernel(N, V, D, per_w)
    out = sc(t.reshape(N), prev.reshape(N), scale_vec, embed_pairs)
    return out.reshape(B, S, D)
